# baseline (speedup 1.0000x reference)
import functools

import jax
import jax.numpy as jnp
from jax import lax
from jax.experimental import pallas as pl
from jax.experimental.pallas import tpu as pltpu
from jax.experimental.pallas import tpu_sc as plsc

H = W = 200
C = 128
N = 1024
OH, OW = 8, 32
S = OH * OW
NWORKERS = 32
BOXES_PER_WORKER = N // NWORKERS
CHUNK = 64
NCH = S // CHUNK


def _coords_body(boxes_ref, idx_ref, w_ref):
    b = boxes_ref[...]
    left = b[:, 0:1]
    top = b[:, 1:2]
    right = b[:, 2:3]
    bottom = b[:, 3:4]
    width = right - left
    height = bottom - top
    ratio = (OW / OH) * height / width
    update = ratio > 1.0
    pad = width * (ratio - 1.0) * 0.5
    nl = jnp.where(update, left - pad, left)
    nr = jnp.where(update, right + pad, right)
    bw = (nr - nl) / OW
    bh = height / OH

    col = lax.broadcasted_iota(jnp.int32, (N, S), 1)
    ow = (col % OW).astype(jnp.float32)
    oh = (col // OW).astype(jnp.float32)
    xs = nl + (ow + 0.5) * bw
    ys = top + (oh + 0.5) * bh
    x0f = jnp.floor(xs)
    y0f = jnp.floor(ys)
    lx = xs - x0f
    ly = ys - y0f
    x0 = jnp.clip(x0f.astype(jnp.int32), 0, W - 1)
    x1 = jnp.minimum(x0 + 1, W - 1)
    y0 = jnp.clip(y0f.astype(jnp.int32), 0, H - 1)
    y1 = jnp.minimum(y0 + 1, H - 1)
    i00 = y0 * W + x0
    i01 = y0 * W + x1
    i10 = y1 * W + x0
    i11 = y1 * W + x1

    dp = (ratio - 1.0) / ratio * (OW / 2)
    keep = (ow >= dp) & (ow < (OW - dp))
    m = jnp.where(update & ~keep, 0.0, 1.0)
    w00 = (1.0 - ly) * (1.0 - lx) * m
    w01 = (1.0 - ly) * lx * m
    w10 = ly * (1.0 - lx) * m
    w11 = ly * lx * m

    for j in range(8):
        corner = j // 2
        half = slice((j % 2) * 128, (j % 2 + 1) * 128)
        idx_ref[:, j, :] = (i00, i01, i10, i11)[corner][:, half]
        w_ref[:, j, :] = (w00, w01, w10, w11)[corner][:, half]


def _make_coords(boxes):
    return pl.pallas_call(
        _coords_body,
        out_shape=(
            jax.ShapeDtypeStruct((N, 8, 128), jnp.int32),
            jax.ShapeDtypeStruct((N, 8, 128), jnp.float32),
        ),
    )(boxes)


def _sc_body(idx_hbm, w_hbm, table_hbm, out_hbm,
             w_v, idx_v, ga0, ga1, ga2, ga3, gb0, gb1, gb2, gb3, obuf,
             gsem_a, gsem_b, wsem, osem):
    wid = lax.axis_index("s") * 2 + lax.axis_index("c")
    gsets = ((ga0, ga1, ga2, ga3, gsem_a), (gb0, gb1, gb2, gb3, gsem_b))
    b0 = wid * BOXES_PER_WORKER

    for j in range(8):
        pltpu.sync_copy(w_hbm.at[b0, j], w_v.at[pl.ds(j * 128, 128)])
    pltpu.sync_copy(idx_hbm.at[b0], idx_v.at[0])

    @pl.loop(0, BOXES_PER_WORKER)
    def _box(k):
        b = b0 + k
        nb = lax.rem(k, 2)
        nbn = lax.rem(k + 1, 2)

        def issue(ch):
            gset = gsets[ch % 2]
            return [
                pltpu.async_copy(
                    table_hbm.at[idx_v.at[nb, c2 * 2 + ch // 2,
                                          pl.ds((ch % 2) * CHUNK, CHUNK)]],
                    gset[c2], gset[4])
                for c2 in range(4)
            ]

        cps_prev = issue(0)
        bn = jnp.minimum(b + 1, N - 1)
        for j in range(8):
            pltpu.async_copy(w_hbm.at[bn, j],
                             w_v.at[pl.ds(nbn * 1024 + j * 128, 128)], wsem)
        pltpu.async_copy(idx_hbm.at[bn], idx_v.at[nbn], wsem)

        @pl.when(k > 0)
        def _():
            for j in range(8):
                pltpu.make_async_copy(w_hbm.at[b, j],
                                      w_v.at[pl.ds(j * 128, 128)], wsem).wait()
            pltpu.make_async_copy(idx_hbm.at[b], idx_v.at[0], wsem).wait()

        for ch in range(NCH):
            cps_next = issue(ch + 1) if ch + 1 < NCH else None
            for cp in cps_prev:
                cp.wait()
            cps_prev = cps_next
            gset = gsets[ch % 2]
            g0, g1, g2, g3 = gset[0], gset[1], gset[2], gset[3]
            ob = ch % 2

            if ch < 2:
                @pl.when(k > 0)
                def _():
                    pltpu.make_async_copy(out_hbm.at[b, pl.ds(0, CHUNK)],
                                          obuf.at[ob], osem).wait()
            else:
                pltpu.make_async_copy(out_hbm.at[b, pl.ds(0, CHUNK)],
                                      obuf.at[ob], osem).wait()

            @plsc.parallel_loop(0, CHUNK, unroll=8)
            def _samp(sl):
                s = ch * CHUNK + sl
                wb = nb * 1024 + s
                w00v = plsc.load_gather(w_v, [jnp.full((16,), 0 * S, jnp.int32) + wb])
                w01v = plsc.load_gather(w_v, [jnp.full((16,), 1 * S, jnp.int32) + wb])
                w10v = plsc.load_gather(w_v, [jnp.full((16,), 2 * S, jnp.int32) + wb])
                w11v = plsc.load_gather(w_v, [jnp.full((16,), 3 * S, jnp.int32) + wb])
                for kb in range(C // 16):
                    o = pl.ds(kb * 16, 16)
                    a00 = g0[sl, o]
                    a01 = g1[sl, o]
                    a10 = g2[sl, o]
                    a11 = g3[sl, o]
                    acc = (w00v * a00 + w01v * a01) + (w10v * a10 + w11v * a11)
                    obuf[ob, sl, o] = acc

            pltpu.async_copy(obuf.at[ob],
                             out_hbm.at[b, pl.ds(ch * CHUNK, CHUNK)], osem)

    pltpu.make_async_copy(out_hbm.at[b0, pl.ds(0, CHUNK)], obuf.at[0], osem).wait()
    pltpu.make_async_copy(out_hbm.at[b0, pl.ds(0, CHUNK)], obuf.at[1], osem).wait()
    for j in range(8):
        pltpu.make_async_copy(w_hbm.at[b0, j],
                              w_v.at[pl.ds(j * 128, 128)], wsem).wait()
    pltpu.make_async_copy(idx_hbm.at[b0], idx_v.at[0], wsem).wait()


def _sc_pool(idx, w, table):
    mesh = plsc.VectorSubcoreMesh(core_axis_name="c", subcore_axis_name="s")
    return pl.kernel(
        _sc_body,
        out_type=jax.ShapeDtypeStruct((N, S, C), jnp.float32),
        mesh=mesh,
        compiler_params=pltpu.CompilerParams(needs_layout_passes=False),
        scratch_types=[
            pltpu.VMEM((2 * 1024,), jnp.float32),
            pltpu.VMEM((2, 8, 128), jnp.int32),
            pltpu.VMEM((CHUNK, C), jnp.float32),
            pltpu.VMEM((CHUNK, C), jnp.float32),
            pltpu.VMEM((CHUNK, C), jnp.float32),
            pltpu.VMEM((CHUNK, C), jnp.float32),
            pltpu.VMEM((CHUNK, C), jnp.float32),
            pltpu.VMEM((CHUNK, C), jnp.float32),
            pltpu.VMEM((CHUNK, C), jnp.float32),
            pltpu.VMEM((CHUNK, C), jnp.float32),
            pltpu.VMEM((2, CHUNK, C), jnp.float32),
            pltpu.SemaphoreType.DMA,
            pltpu.SemaphoreType.DMA,
            pltpu.SemaphoreType.DMA,
            pltpu.SemaphoreType.DMA,
        ],
    )(idx, w, table)


def kernel(features, boxes):
    table = features[0].transpose(1, 2, 0).reshape(H * W, C)
    idx, w = _make_coords(boxes)
    out = _sc_pool(idx, w, table)
    return out.reshape(N, OH, OW, C).transpose(0, 3, 1, 2)


# speedup vs baseline: 1.1659x; 1.1659x over previous
import functools

import jax
import jax.numpy as jnp
from jax import lax
from jax.experimental import pallas as pl
from jax.experimental.pallas import tpu as pltpu
from jax.experimental.pallas import tpu_sc as plsc

H = W = 200
C = 128
N = 1024
OH, OW = 8, 32
S = OH * OW
NWORKERS = 32
BOXES_PER_WORKER = N // NWORKERS
CHUNK = 64
NCH = S // CHUNK


def _coords_body(boxes_ref, idx_ref, w_ref):
    b = boxes_ref[...]
    left = b[:, 0:1]
    top = b[:, 1:2]
    right = b[:, 2:3]
    bottom = b[:, 3:4]
    width = right - left
    height = bottom - top
    ratio = (OW / OH) * height / width
    update = ratio > 1.0
    pad = width * (ratio - 1.0) * 0.5
    nl = jnp.where(update, left - pad, left)
    nr = jnp.where(update, right + pad, right)
    bw = (nr - nl) / OW
    bh = height / OH

    col = lax.broadcasted_iota(jnp.int32, (N, S), 1)
    ow = (col % OW).astype(jnp.float32)
    oh = (col // OW).astype(jnp.float32)
    xs = nl + (ow + 0.5) * bw
    ys = top + (oh + 0.5) * bh
    x0f = jnp.floor(xs)
    y0f = jnp.floor(ys)
    lx = xs - x0f
    ly = ys - y0f
    x0 = jnp.clip(x0f.astype(jnp.int32), 0, W - 1)
    x1 = jnp.minimum(x0 + 1, W - 1)
    y0 = jnp.clip(y0f.astype(jnp.int32), 0, H - 1)
    y1 = jnp.minimum(y0 + 1, H - 1)
    i00 = y0 * W + x0
    i01 = y0 * W + x1
    i10 = y1 * W + x0
    i11 = y1 * W + x1

    dp = (ratio - 1.0) / ratio * (OW / 2)
    keep = (ow >= dp) & (ow < (OW - dp))
    m = jnp.where(update & ~keep, 0.0, 1.0)
    w00 = (1.0 - ly) * (1.0 - lx) * m
    w01 = (1.0 - ly) * lx * m
    w10 = ly * (1.0 - lx) * m
    w11 = ly * lx * m

    for j in range(8):
        corner = j // 2
        half = slice((j % 2) * 128, (j % 2 + 1) * 128)
        idx_ref[:, j, :] = (i00, i01, i10, i11)[corner][:, half]
        w_ref[:, j, :] = (w00, w01, w10, w11)[corner][:, half]


def _make_coords(boxes):
    return pl.pallas_call(
        _coords_body,
        out_shape=(
            jax.ShapeDtypeStruct((N, 8, 128), jnp.int32),
            jax.ShapeDtypeStruct((N, 8, 128), jnp.float32),
        ),
    )(boxes)


def _sc_body(idx_hbm, w_hbm, table_hbm, out_hbm,
             w_v, idx_v, ga0, ga1, ga2, ga3, gb0, gb1, gb2, gb3, obuf,
             gsem_a, gsem_b, wsem, osem):
    wid = lax.axis_index("s") * 2 + lax.axis_index("c")
    gsets = ((ga0, ga1, ga2, ga3, gsem_a), (gb0, gb1, gb2, gb3, gsem_b))
    b0 = wid * BOXES_PER_WORKER

    for j in range(8):
        pltpu.sync_copy(w_hbm.at[b0, j], w_v.at[pl.ds(j * 128, 128)])
    pltpu.sync_copy(idx_hbm.at[b0], idx_v.at[0])

    @pl.loop(0, BOXES_PER_WORKER)
    def _box(k):
        b = b0 + k
        nb = lax.rem(k, 2)
        nbn = lax.rem(k + 1, 2)

        def issue(ch):
            gset = gsets[ch % 2]
            return [
                pltpu.async_copy(
                    table_hbm.at[idx_v.at[nb, c2 * 2 + ch // 2,
                                          pl.ds((ch % 2) * CHUNK, CHUNK)]],
                    gset[c2], gset[4])
                for c2 in range(4)
            ]

        cps_prev = issue(0)
        bn = jnp.minimum(b + 1, N - 1)
        for j in range(8):
            pltpu.async_copy(w_hbm.at[bn, j],
                             w_v.at[pl.ds(nbn * 1024 + j * 128, 128)], wsem)
        pltpu.async_copy(idx_hbm.at[bn], idx_v.at[nbn], wsem)

        @pl.when(k > 0)
        def _():
            for j in range(8):
                pltpu.make_async_copy(w_hbm.at[b, j],
                                      w_v.at[pl.ds(j * 128, 128)], wsem).wait()
            pltpu.make_async_copy(idx_hbm.at[b], idx_v.at[0], wsem).wait()

        for ch in range(NCH):
            cps_next = issue(ch + 1) if ch + 1 < NCH else None
            for cp in cps_prev:
                cp.wait()
            cps_prev = cps_next
            gset = gsets[ch % 2]
            g0, g1, g2, g3 = gset[0], gset[1], gset[2], gset[3]
            ob = ch % 2

            if ch < 2:
                @pl.when(k > 0)
                def _():
                    pltpu.make_async_copy(out_hbm.at[b, pl.ds(0, CHUNK)],
                                          obuf.at[ob], osem).wait()
            else:
                pltpu.make_async_copy(out_hbm.at[b, pl.ds(0, CHUNK)],
                                      obuf.at[ob], osem).wait()

            @plsc.parallel_loop(0, CHUNK, unroll=4)
            def _samp(sl):
                s = ch * CHUNK + sl
                wb = nb * 1024 + s
                w00v = plsc.load_gather(w_v, [jnp.full((16,), 0 * S, jnp.int32) + wb])
                w01v = plsc.load_gather(w_v, [jnp.full((16,), 1 * S, jnp.int32) + wb])
                w10v = plsc.load_gather(w_v, [jnp.full((16,), 2 * S, jnp.int32) + wb])
                w11v = plsc.load_gather(w_v, [jnp.full((16,), 3 * S, jnp.int32) + wb])
                for kb in range(C // 16):
                    o = pl.ds(kb * 16, 16)
                    a00 = g0[sl, o]
                    a01 = g1[sl, o]
                    a10 = g2[sl, o]
                    a11 = g3[sl, o]
                    acc = (w00v * a00 + w01v * a01) + (w10v * a10 + w11v * a11)
                    obuf[ob, sl, o] = acc

            pltpu.async_copy(obuf.at[ob],
                             out_hbm.at[b, pl.ds(ch * CHUNK, CHUNK)], osem)

    pltpu.make_async_copy(out_hbm.at[b0, pl.ds(0, CHUNK)], obuf.at[0], osem).wait()
    pltpu.make_async_copy(out_hbm.at[b0, pl.ds(0, CHUNK)], obuf.at[1], osem).wait()
    for j in range(8):
        pltpu.make_async_copy(w_hbm.at[b0, j],
                              w_v.at[pl.ds(j * 128, 128)], wsem).wait()
    pltpu.make_async_copy(idx_hbm.at[b0], idx_v.at[0], wsem).wait()


def _sc_pool(idx, w, table):
    mesh = plsc.VectorSubcoreMesh(core_axis_name="c", subcore_axis_name="s")
    return pl.kernel(
        _sc_body,
        out_type=jax.ShapeDtypeStruct((N, S, C), jnp.float32),
        mesh=mesh,
        compiler_params=pltpu.CompilerParams(needs_layout_passes=False),
        scratch_types=[
            pltpu.VMEM((2 * 1024,), jnp.float32),
            pltpu.VMEM((2, 8, 128), jnp.int32),
            pltpu.VMEM((CHUNK, C), jnp.float32),
            pltpu.VMEM((CHUNK, C), jnp.float32),
            pltpu.VMEM((CHUNK, C), jnp.float32),
            pltpu.VMEM((CHUNK, C), jnp.float32),
            pltpu.VMEM((CHUNK, C), jnp.float32),
            pltpu.VMEM((CHUNK, C), jnp.float32),
            pltpu.VMEM((CHUNK, C), jnp.float32),
            pltpu.VMEM((CHUNK, C), jnp.float32),
            pltpu.VMEM((2, CHUNK, C), jnp.float32),
            pltpu.SemaphoreType.DMA,
            pltpu.SemaphoreType.DMA,
            pltpu.SemaphoreType.DMA,
            pltpu.SemaphoreType.DMA,
        ],
    )(idx, w, table)


def kernel(features, boxes):
    table = features[0].transpose(1, 2, 0).reshape(H * W, C)
    idx, w = _make_coords(boxes)
    out = _sc_pool(idx, w, table)
    return out.reshape(N, OH, OW, C).transpose(0, 3, 1, 2)


# speedup vs baseline: 1.5130x; 1.2977x over previous
import functools

import jax
import jax.numpy as jnp
from jax import lax
from jax.experimental import pallas as pl
from jax.experimental.pallas import tpu as pltpu
from jax.experimental.pallas import tpu_sc as plsc

H = W = 200
C = 128
N = 1024
OH, OW = 8, 32
S = OH * OW
NWORKERS = 32
BOXES_PER_WORKER = N // NWORKERS
CHUNK = 128
NCH = S // CHUNK


def _coords_body(boxes_ref, idx_ref, w_ref):
    b = boxes_ref[...]
    left = b[:, 0:1]
    top = b[:, 1:2]
    right = b[:, 2:3]
    bottom = b[:, 3:4]
    width = right - left
    height = bottom - top
    ratio = (OW / OH) * height / width
    update = ratio > 1.0
    pad = width * (ratio - 1.0) * 0.5
    nl = jnp.where(update, left - pad, left)
    nr = jnp.where(update, right + pad, right)
    bw = (nr - nl) / OW
    bh = height / OH

    col = lax.broadcasted_iota(jnp.int32, (N, S), 1)
    ow = (col % OW).astype(jnp.float32)
    oh = (col // OW).astype(jnp.float32)
    xs = nl + (ow + 0.5) * bw
    ys = top + (oh + 0.5) * bh
    x0f = jnp.floor(xs)
    y0f = jnp.floor(ys)
    lx = xs - x0f
    ly = ys - y0f
    x0 = jnp.clip(x0f.astype(jnp.int32), 0, W - 1)
    x1 = jnp.minimum(x0 + 1, W - 1)
    y0 = jnp.clip(y0f.astype(jnp.int32), 0, H - 1)
    y1 = jnp.minimum(y0 + 1, H - 1)
    i00 = y0 * W + x0
    i10 = y1 * W + x0

    dp = (ratio - 1.0) / ratio * (OW / 2)
    keep = (ow >= dp) & (ow < (OW - dp))
    m = jnp.where(update & ~keep, 0.0, 1.0)
    w00 = (1.0 - ly) * (1.0 - lx) * m
    w01 = (1.0 - ly) * lx * m
    w10 = ly * (1.0 - lx) * m
    w11 = ly * lx * m
    clipped = x1 == x0
    w00 = jnp.where(clipped, w00 + w01, w00)
    w01 = jnp.where(clipped, 0.0, w01)
    w10 = jnp.where(clipped, w10 + w11, w10)
    w11 = jnp.where(clipped, 0.0, w11)

    for j in range(8):
        half = slice((j % 2) * 128, (j % 2 + 1) * 128)
        idx_ref[:, j, :] = (i00, i10)[(j // 2) % 2][:, half]
        w_ref[:, j, :] = (w00, w01, w10, w11)[j // 2][:, half]


def _make_coords(boxes):
    return pl.pallas_call(
        _coords_body,
        out_shape=(
            jax.ShapeDtypeStruct((N, 8, 128), jnp.int32),
            jax.ShapeDtypeStruct((N, 8, 128), jnp.float32),
        ),
    )(boxes)


def _sc_body(idx_hbm, w_hbm, table_hbm, out_hbm,
             w_v, idx_v, ga0, ga1, gb0, gb1, obuf,
             gsem_a, gsem_b, wsem, osem):
    wid = lax.axis_index("s") * 2 + lax.axis_index("c")
    gsets = ((ga0, ga1, gsem_a), (gb0, gb1, gsem_b))
    b0 = wid * BOXES_PER_WORKER

    for j in range(8):
        pltpu.sync_copy(w_hbm.at[b0, j], w_v.at[pl.ds(j * 128, 128)])
    pltpu.sync_copy(idx_hbm.at[b0], idx_v.at[0])

    @pl.loop(0, BOXES_PER_WORKER)
    def _box(k):
        b = b0 + k
        nb = lax.rem(k, 2)
        nbn = lax.rem(k + 1, 2)

        def issue(ch):
            gset = gsets[ch % 2]
            return [
                pltpu.async_copy(
                    table_hbm.at[idx_v.at[nb, c2 * 2 + ch]],
                    gset[c2], gset[2])
                for c2 in range(2)
            ]

        cps_prev = issue(0)
        bn = jnp.minimum(b + 1, N - 1)
        for j in range(8):
            pltpu.async_copy(w_hbm.at[bn, j],
                             w_v.at[pl.ds(nbn * 1024 + j * 128, 128)], wsem)
        pltpu.async_copy(idx_hbm.at[bn], idx_v.at[nbn], wsem)

        @pl.when(k > 0)
        def _():
            for j in range(8):
                pltpu.make_async_copy(w_hbm.at[b, j],
                                      w_v.at[pl.ds(j * 128, 128)], wsem).wait()
            pltpu.make_async_copy(idx_hbm.at[b], idx_v.at[0], wsem).wait()

        for ch in range(NCH):
            cps_next = issue(ch + 1) if ch + 1 < NCH else None
            for cp in cps_prev:
                cp.wait()
            cps_prev = cps_next
            gset = gsets[ch % 2]
            g0, g1 = gset[0], gset[1]
            ob = ch % 2

            if ch < 2:
                @pl.when(k > 0)
                def _():
                    pltpu.make_async_copy(out_hbm.at[b, pl.ds(0, CHUNK)],
                                          obuf.at[ob], osem).wait()
            else:
                pltpu.make_async_copy(out_hbm.at[b, pl.ds(0, CHUNK)],
                                      obuf.at[ob], osem).wait()

            @plsc.parallel_loop(0, CHUNK, unroll=4)
            def _samp(sl):
                s = ch * CHUNK + sl
                wb = nb * 1024 + s
                w00v = plsc.load_gather(w_v, [jnp.full((16,), 0 * S, jnp.int32) + wb])
                w01v = plsc.load_gather(w_v, [jnp.full((16,), 1 * S, jnp.int32) + wb])
                w10v = plsc.load_gather(w_v, [jnp.full((16,), 2 * S, jnp.int32) + wb])
                w11v = plsc.load_gather(w_v, [jnp.full((16,), 3 * S, jnp.int32) + wb])
                wa = plsc.pack(w00v, w01v, format=plsc.PackFormat.INTERLEAVED)
                wb = plsc.pack(w10v, w11v, format=plsc.PackFormat.INTERLEAVED)
                for kb in range(C // 16):
                    o = pl.ds(kb * 16, 16)
                    a0 = plsc.bitcast(g0[sl, o], jnp.bfloat16)
                    a1 = plsc.bitcast(g1[sl, o], jnp.bfloat16)
                    acc = wa * a0 + wb * a1
                    ev, od = plsc.unpack(acc, format=plsc.PackFormat.INTERLEAVED,
                                         preferred_element_type=jnp.float32)
                    obuf[ob, sl, o] = ev + od

            pltpu.async_copy(obuf.at[ob],
                             out_hbm.at[b, pl.ds(ch * CHUNK, CHUNK)], osem)

    pltpu.make_async_copy(out_hbm.at[b0, pl.ds(0, CHUNK)], obuf.at[0], osem).wait()
    pltpu.make_async_copy(out_hbm.at[b0, pl.ds(0, CHUNK)], obuf.at[1], osem).wait()
    for j in range(8):
        pltpu.make_async_copy(w_hbm.at[b0, j],
                              w_v.at[pl.ds(j * 128, 128)], wsem).wait()
    pltpu.make_async_copy(idx_hbm.at[b0], idx_v.at[0], wsem).wait()


def _sc_pool(idx, w, table):
    mesh = plsc.VectorSubcoreMesh(core_axis_name="c", subcore_axis_name="s")
    return pl.kernel(
        _sc_body,
        out_type=jax.ShapeDtypeStruct((N, S, C), jnp.float32),
        mesh=mesh,
        compiler_params=pltpu.CompilerParams(needs_layout_passes=False),
        scratch_types=[
            pltpu.VMEM((2 * 1024,), jnp.float32),
            pltpu.VMEM((2, 8, 128), jnp.int32),
            pltpu.VMEM((CHUNK, C), jnp.float32),
            pltpu.VMEM((CHUNK, C), jnp.float32),
            pltpu.VMEM((CHUNK, C), jnp.float32),
            pltpu.VMEM((CHUNK, C), jnp.float32),
            pltpu.VMEM((2, CHUNK, C), jnp.float32),
            pltpu.SemaphoreType.DMA,
            pltpu.SemaphoreType.DMA,
            pltpu.SemaphoreType.DMA,
            pltpu.SemaphoreType.DMA,
        ],
    )(idx, w, table)


def kernel(features, boxes):
    table = features[0].transpose(1, 2, 0).reshape(H * W, C)
    tb = table.astype(jnp.bfloat16)
    tn = jnp.concatenate([tb[1:], tb[-1:]], axis=0)
    table = lax.bitcast_convert_type(
        jnp.stack([tb, tn], axis=-1), jnp.float32)
    idx, w = _make_coords(boxes)
    out = _sc_pool(idx, w, table)
    return out.reshape(N, OH, OW, C).transpose(0, 3, 1, 2)


# speedup vs baseline: 1.5261x; 1.0086x over previous
import functools

import jax
import jax.numpy as jnp
from jax import lax
from jax.experimental import pallas as pl
from jax.experimental.pallas import tpu as pltpu
from jax.experimental.pallas import tpu_sc as plsc

H = W = 200
C = 128
N = 1024
OH, OW = 8, 32
S = OH * OW
NWORKERS = 32
BOXES_PER_WORKER = N // NWORKERS
CHUNK = 128
NCH = S // CHUNK


_PACK_R = 400


def _pack_body(a_ref, b_ref, t_ref):
    a = a_ref[...]
    nxt = jnp.concatenate([a[1:], b_ref[0:1]], axis=0)
    lo = lax.bitcast_convert_type(a.astype(jnp.bfloat16),
                                  jnp.uint16).astype(jnp.uint32)
    hi = lax.bitcast_convert_type(nxt.astype(jnp.bfloat16),
                                  jnp.uint16).astype(jnp.uint32)
    t_ref[...] = lax.bitcast_convert_type(lo | (hi << 16), jnp.float32)


def _make_packed_table(t2d):
    g = (H * W) // _PACK_R
    return pl.pallas_call(
        _pack_body,
        grid=(g,),
        in_specs=[
            pl.BlockSpec((_PACK_R, C), lambda i: (i, 0)),
            pl.BlockSpec((_PACK_R, C), lambda i: (jnp.minimum(i + 1, g - 1), 0)),
        ],
        out_specs=pl.BlockSpec((_PACK_R, C), lambda i: (i, 0)),
        out_shape=jax.ShapeDtypeStruct((H * W, C), jnp.float32),
    )(t2d, t2d)


def _coords_body(boxes_ref, idx_ref, w_ref):
    b = boxes_ref[...]
    left = b[:, 0:1]
    top = b[:, 1:2]
    right = b[:, 2:3]
    bottom = b[:, 3:4]
    width = right - left
    height = bottom - top
    ratio = (OW / OH) * height / width
    update = ratio > 1.0
    pad = width * (ratio - 1.0) * 0.5
    nl = jnp.where(update, left - pad, left)
    nr = jnp.where(update, right + pad, right)
    bw = (nr - nl) / OW
    bh = height / OH

    col = lax.broadcasted_iota(jnp.int32, (N, S), 1)
    ow = (col % OW).astype(jnp.float32)
    oh = (col // OW).astype(jnp.float32)
    xs = nl + (ow + 0.5) * bw
    ys = top + (oh + 0.5) * bh
    x0f = jnp.floor(xs)
    y0f = jnp.floor(ys)
    lx = xs - x0f
    ly = ys - y0f
    x0 = jnp.clip(x0f.astype(jnp.int32), 0, W - 1)
    x1 = jnp.minimum(x0 + 1, W - 1)
    y0 = jnp.clip(y0f.astype(jnp.int32), 0, H - 1)
    y1 = jnp.minimum(y0 + 1, H - 1)
    i00 = y0 * W + x0
    i10 = y1 * W + x0

    dp = (ratio - 1.0) / ratio * (OW / 2)
    keep = (ow >= dp) & (ow < (OW - dp))
    m = jnp.where(update & ~keep, 0.0, 1.0)
    w00 = (1.0 - ly) * (1.0 - lx) * m
    w01 = (1.0 - ly) * lx * m
    w10 = ly * (1.0 - lx) * m
    w11 = ly * lx * m
    clipped = x1 == x0
    w00 = jnp.where(clipped, w00 + w01, w00)
    w01 = jnp.where(clipped, 0.0, w01)
    w10 = jnp.where(clipped, w10 + w11, w10)
    w11 = jnp.where(clipped, 0.0, w11)

    def packw(lo, hi):
        lo_u = lax.bitcast_convert_type(lo.astype(jnp.bfloat16),
                                        jnp.uint16).astype(jnp.uint32)
        hi_u = lax.bitcast_convert_type(hi.astype(jnp.bfloat16),
                                        jnp.uint16).astype(jnp.uint32)
        return lax.bitcast_convert_type(lo_u | (hi_u << 16), jnp.float32)

    wa = packw(w00, w01)
    wb = packw(w10, w11)

    for j in range(8):
        half = slice((j % 2) * 128, (j % 2 + 1) * 128)
        idx_ref[:, j, :] = (i00, i10)[(j // 2) % 2][:, half]
        w_ref[:, j, :] = (wa, wb)[(j // 2) % 2][:, half]


def _make_coords(boxes):
    return pl.pallas_call(
        _coords_body,
        out_shape=(
            jax.ShapeDtypeStruct((N, 8, 128), jnp.int32),
            jax.ShapeDtypeStruct((N, 8, 128), jnp.float32),
        ),
    )(boxes)


def _sc_body(idx_hbm, w_hbm, table_hbm, out_hbm,
             w_v, idx_v, ga0, ga1, gb0, gb1, obuf,
             gsem_a, gsem_b, wsem, osem):
    wid = lax.axis_index("s") * 2 + lax.axis_index("c")
    gsets = ((ga0, ga1, gsem_a), (gb0, gb1, gsem_b))
    b0 = wid * BOXES_PER_WORKER

    for j in range(4):
        pltpu.sync_copy(w_hbm.at[b0, j], w_v.at[pl.ds(j * 128, 128)])
    pltpu.sync_copy(idx_hbm.at[b0], idx_v.at[0])

    @pl.loop(0, BOXES_PER_WORKER)
    def _box(k):
        b = b0 + k
        nb = lax.rem(k, 2)
        nbn = lax.rem(k + 1, 2)

        def issue(ch):
            gset = gsets[ch % 2]
            return [
                pltpu.async_copy(
                    table_hbm.at[idx_v.at[nb, c2 * 2 + ch]],
                    gset[c2], gset[2])
                for c2 in range(2)
            ]

        cps_prev = issue(0)
        bn = jnp.minimum(b + 1, N - 1)
        for j in range(4):
            pltpu.async_copy(w_hbm.at[bn, j],
                             w_v.at[pl.ds(nbn * 512 + j * 128, 128)], wsem)
        pltpu.async_copy(idx_hbm.at[bn], idx_v.at[nbn], wsem)

        @pl.when(k > 0)
        def _():
            for j in range(4):
                pltpu.make_async_copy(w_hbm.at[b, j],
                                      w_v.at[pl.ds(j * 128, 128)], wsem).wait()
            pltpu.make_async_copy(idx_hbm.at[b], idx_v.at[0], wsem).wait()

        for ch in range(NCH):
            cps_next = issue(ch + 1) if ch + 1 < NCH else None
            for cp in cps_prev:
                cp.wait()
            cps_prev = cps_next
            gset = gsets[ch % 2]
            g0, g1 = gset[0], gset[1]
            ob = ch % 2

            if ch < 2:
                @pl.when(k > 0)
                def _():
                    pltpu.make_async_copy(out_hbm.at[b, pl.ds(0, CHUNK)],
                                          obuf.at[ob], osem).wait()
            else:
                pltpu.make_async_copy(out_hbm.at[b, pl.ds(0, CHUNK)],
                                      obuf.at[ob], osem).wait()

            @plsc.parallel_loop(0, CHUNK, unroll=4)
            def _samp(sl):
                s = ch * CHUNK + sl
                woff = nb * 512 + s
                waw = plsc.load_gather(w_v, [jnp.full((16,), 0, jnp.int32) + woff])
                wbw = plsc.load_gather(w_v, [jnp.full((16,), S, jnp.int32) + woff])
                wa = plsc.bitcast(waw, jnp.bfloat16)
                wb2 = plsc.bitcast(wbw, jnp.bfloat16)
                for kb in range(C // 16):
                    o = pl.ds(kb * 16, 16)
                    a0 = plsc.bitcast(g0[sl, o], jnp.bfloat16)
                    a1 = plsc.bitcast(g1[sl, o], jnp.bfloat16)
                    acc = wa * a0 + wb2 * a1
                    ev, od = plsc.unpack(acc, format=plsc.PackFormat.INTERLEAVED,
                                         preferred_element_type=jnp.float32)
                    obuf[ob, sl, o] = ev + od

            pltpu.async_copy(obuf.at[ob],
                             out_hbm.at[b, pl.ds(ch * CHUNK, CHUNK)], osem)

    pltpu.make_async_copy(out_hbm.at[b0, pl.ds(0, CHUNK)], obuf.at[0], osem).wait()
    pltpu.make_async_copy(out_hbm.at[b0, pl.ds(0, CHUNK)], obuf.at[1], osem).wait()
    for j in range(4):
        pltpu.make_async_copy(w_hbm.at[b0, j],
                              w_v.at[pl.ds(j * 128, 128)], wsem).wait()
    pltpu.make_async_copy(idx_hbm.at[b0], idx_v.at[0], wsem).wait()


def _sc_pool(idx, w, table):
    mesh = plsc.VectorSubcoreMesh(core_axis_name="c", subcore_axis_name="s")
    return pl.kernel(
        _sc_body,
        out_type=jax.ShapeDtypeStruct((N, S, C), jnp.float32),
        mesh=mesh,
        compiler_params=pltpu.CompilerParams(needs_layout_passes=False),
        scratch_types=[
            pltpu.VMEM((2 * 512,), jnp.float32),
            pltpu.VMEM((2, 8, 128), jnp.int32),
            pltpu.VMEM((CHUNK, C), jnp.float32),
            pltpu.VMEM((CHUNK, C), jnp.float32),
            pltpu.VMEM((CHUNK, C), jnp.float32),
            pltpu.VMEM((CHUNK, C), jnp.float32),
            pltpu.VMEM((2, CHUNK, C), jnp.float32),
            pltpu.SemaphoreType.DMA,
            pltpu.SemaphoreType.DMA,
            pltpu.SemaphoreType.DMA,
            pltpu.SemaphoreType.DMA,
        ],
    )(idx, w, table)


def kernel(features, boxes):
    t2d = features[0].transpose(1, 2, 0).reshape(H * W, C)
    table = _make_packed_table(t2d)
    idx, w = _make_coords(boxes)
    out = _sc_pool(idx, w, table)
    return out.reshape(N, OH, OW, C).transpose(0, 3, 1, 2)


# speedup vs baseline: 1.7699x; 1.1598x over previous
import functools

import jax
import jax.numpy as jnp
from jax import lax
from jax.experimental import pallas as pl
from jax.experimental.pallas import tpu as pltpu
from jax.experimental.pallas import tpu_sc as plsc

H = W = 200
C = 128
N = 1024
OH, OW = 8, 32
S = OH * OW
NWORKERS = 32
BOXES_PER_WORKER = N // NWORKERS
CHUNK = 128
NCH = S // CHUNK


_PACK_R = 400


def _pack_body(a_ref, b_ref, t_ref):
    a = a_ref[...]
    nxt = jnp.concatenate([a[1:], b_ref[0:1]], axis=0)
    lo = lax.bitcast_convert_type(a.astype(jnp.bfloat16),
                                  jnp.uint16).astype(jnp.uint32)
    hi = lax.bitcast_convert_type(nxt.astype(jnp.bfloat16),
                                  jnp.uint16).astype(jnp.uint32)
    t_ref[...] = lax.bitcast_convert_type(lo | (hi << 16), jnp.float32)


def _make_packed_table(t2d):
    g = (H * W) // _PACK_R
    return pl.pallas_call(
        _pack_body,
        grid=(g,),
        in_specs=[
            pl.BlockSpec((_PACK_R, C), lambda i: (i, 0)),
            pl.BlockSpec((_PACK_R, C), lambda i: (jnp.minimum(i + 1, g - 1), 0)),
        ],
        out_specs=pl.BlockSpec((_PACK_R, C), lambda i: (i, 0)),
        out_shape=jax.ShapeDtypeStruct((H * W, C), jnp.float32),
    )(t2d, t2d)


def _coords_body(boxes_ref, idx_ref, w_ref):
    b = boxes_ref[...]
    left = b[:, 0:1]
    top = b[:, 1:2]
    right = b[:, 2:3]
    bottom = b[:, 3:4]
    width = right - left
    height = bottom - top
    ratio = (OW / OH) * height / width
    update = ratio > 1.0
    pad = width * (ratio - 1.0) * 0.5
    nl = jnp.where(update, left - pad, left)
    nr = jnp.where(update, right + pad, right)
    bw = (nr - nl) / OW
    bh = height / OH

    col = lax.broadcasted_iota(jnp.int32, (N, S), 1)
    ow = (col % OW).astype(jnp.float32)
    oh = (col // OW).astype(jnp.float32)
    xs = nl + (ow + 0.5) * bw
    ys = top + (oh + 0.5) * bh
    x0f = jnp.floor(xs)
    y0f = jnp.floor(ys)
    lx = xs - x0f
    ly = ys - y0f
    x0 = jnp.clip(x0f.astype(jnp.int32), 0, W - 1)
    x1 = jnp.minimum(x0 + 1, W - 1)
    y0 = jnp.clip(y0f.astype(jnp.int32), 0, H - 1)
    y1 = jnp.minimum(y0 + 1, H - 1)
    i00 = y0 * W + x0
    i10 = y1 * W + x0

    dp = (ratio - 1.0) / ratio * (OW / 2)
    keep = (ow >= dp) & (ow < (OW - dp))
    m = jnp.where(update & ~keep, 0.0, 1.0)
    w00 = (1.0 - ly) * (1.0 - lx) * m
    w01 = (1.0 - ly) * lx * m
    w10 = ly * (1.0 - lx) * m
    w11 = ly * lx * m
    clipped = x1 == x0
    w00 = jnp.where(clipped, w00 + w01, w00)
    w01 = jnp.where(clipped, 0.0, w01)
    w10 = jnp.where(clipped, w10 + w11, w10)
    w11 = jnp.where(clipped, 0.0, w11)

    def packw(lo, hi):
        lo_u = lax.bitcast_convert_type(lo.astype(jnp.bfloat16),
                                        jnp.uint16).astype(jnp.uint32)
        hi_u = lax.bitcast_convert_type(hi.astype(jnp.bfloat16),
                                        jnp.uint16).astype(jnp.uint32)
        return lax.bitcast_convert_type(lo_u | (hi_u << 16), jnp.float32)

    wa = packw(w00, w01)
    wb = packw(w10, w11)

    for j in range(8):
        half = slice((j % 2) * 128, (j % 2 + 1) * 128)
        idx_ref[:, j, :] = (i00, i10)[(j // 2) % 2][:, half]
        w_ref[:, j, :] = (wa, wb)[(j // 2) % 2][:, half]


def _make_coords(boxes):
    return pl.pallas_call(
        _coords_body,
        out_shape=(
            jax.ShapeDtypeStruct((N, 8, 128), jnp.int32),
            jax.ShapeDtypeStruct((N, 8, 128), jnp.float32),
        ),
    )(boxes)


def _sc_body(idx_hbm, w_hbm, table_hbm, out_hbm,
             w_v, idx_v, ga0, ga1, gb0, gb1, obuf,
             gsem_a, gsem_b, wsem, osem):
    wid = lax.axis_index("s") * 2 + lax.axis_index("c")
    gsets = ((ga0, ga1, gsem_a), (gb0, gb1, gsem_b))
    b0 = wid * BOXES_PER_WORKER

    def issue(slot, ch, gset):
        for c2 in range(2):
            pltpu.async_copy(table_hbm.at[idx_v.at[slot, c2 * 2 + ch]],
                             gset[c2], gset[2])

    def drain_gathers(gset):
        for c2 in range(2):
            pltpu.make_async_copy(table_hbm.at[pl.ds(0, CHUNK)],
                                  gset[c2], gset[2]).wait()

    for j in range(4):
        pltpu.sync_copy(w_hbm.at[b0, j], w_v.at[pl.ds(j * 128, 128)])
    pltpu.sync_copy(idx_hbm.at[b0], idx_v.at[0])
    issue(0, 0, gsets[0])

    @pl.loop(0, BOXES_PER_WORKER)
    def _box(k):
        b = b0 + k
        nb = lax.rem(k, 2)
        nbn = lax.rem(k + 1, 2)

        bn = jnp.minimum(b + 1, N - 1)
        for j in range(4):
            pltpu.async_copy(w_hbm.at[bn, j],
                             w_v.at[pl.ds(nbn * 512 + j * 128, 128)], wsem)
        pltpu.async_copy(idx_hbm.at[bn], idx_v.at[nbn], wsem)

        for ch in range(NCH):
            if ch == 0:
                issue(nb, 1, gsets[1])
            else:
                for j in range(4):
                    pltpu.make_async_copy(
                        w_hbm.at[b, j], w_v.at[pl.ds(j * 128, 128)], wsem).wait()
                pltpu.make_async_copy(idx_hbm.at[b], idx_v.at[0], wsem).wait()
                issue(nbn, 0, gsets[0])
            drain_gathers(gsets[ch])
            gset = gsets[ch % 2]
            g0, g1 = gset[0], gset[1]
            ob = ch % 2

            if ch < 2:
                @pl.when(k > 0)
                def _():
                    pltpu.make_async_copy(out_hbm.at[b, pl.ds(0, CHUNK)],
                                          obuf.at[ob], osem).wait()
            else:
                pltpu.make_async_copy(out_hbm.at[b, pl.ds(0, CHUNK)],
                                      obuf.at[ob], osem).wait()

            @plsc.parallel_loop(0, CHUNK, unroll=4)
            def _samp(sl):
                s = ch * CHUNK + sl
                woff = nb * 512 + s
                waw = plsc.load_gather(w_v, [jnp.full((16,), 0, jnp.int32) + woff])
                wbw = plsc.load_gather(w_v, [jnp.full((16,), S, jnp.int32) + woff])
                wa = plsc.bitcast(waw, jnp.bfloat16)
                wb2 = plsc.bitcast(wbw, jnp.bfloat16)
                for kb in range(C // 16):
                    o = pl.ds(kb * 16, 16)
                    a0 = plsc.bitcast(g0[sl, o], jnp.bfloat16)
                    a1 = plsc.bitcast(g1[sl, o], jnp.bfloat16)
                    acc = wa * a0 + wb2 * a1
                    ev, od = plsc.unpack(acc, format=plsc.PackFormat.INTERLEAVED,
                                         preferred_element_type=jnp.float32)
                    obuf[ob, sl, o] = ev + od

            pltpu.async_copy(obuf.at[ob],
                             out_hbm.at[b, pl.ds(ch * CHUNK, CHUNK)], osem)

    pltpu.make_async_copy(out_hbm.at[b0, pl.ds(0, CHUNK)], obuf.at[0], osem).wait()
    pltpu.make_async_copy(out_hbm.at[b0, pl.ds(0, CHUNK)], obuf.at[1], osem).wait()
    drain_gathers(gsets[0])


def _sc_pool(idx, w, table):
    mesh = plsc.VectorSubcoreMesh(core_axis_name="c", subcore_axis_name="s")
    return pl.kernel(
        _sc_body,
        out_type=jax.ShapeDtypeStruct((N, S, C), jnp.float32),
        mesh=mesh,
        compiler_params=pltpu.CompilerParams(needs_layout_passes=False),
        scratch_types=[
            pltpu.VMEM((2 * 512,), jnp.float32),
            pltpu.VMEM((2, 8, 128), jnp.int32),
            pltpu.VMEM((CHUNK, C), jnp.float32),
            pltpu.VMEM((CHUNK, C), jnp.float32),
            pltpu.VMEM((CHUNK, C), jnp.float32),
            pltpu.VMEM((CHUNK, C), jnp.float32),
            pltpu.VMEM((2, CHUNK, C), jnp.float32),
            pltpu.SemaphoreType.DMA,
            pltpu.SemaphoreType.DMA,
            pltpu.SemaphoreType.DMA,
            pltpu.SemaphoreType.DMA,
        ],
    )(idx, w, table)


def kernel(features, boxes):
    t2d = features[0].transpose(1, 2, 0).reshape(H * W, C)
    table = _make_packed_table(t2d)
    idx, w = _make_coords(boxes)
    out = _sc_pool(idx, w, table)
    return out.reshape(N, OH, OW, C).transpose(0, 3, 1, 2)
